# Initial kernel scaffold; baseline (speedup 1.0000x reference)
import functools

import jax
import jax.numpy as jnp
import numpy as np
from jax import lax
from jax.experimental import pallas as pl
from jax.experimental.pallas import tpu as pltpu
from jax.experimental.pallas import tpu_sc as plsc

N = 10000
D = 256
DH = 128
L = 3
NC = 2
NS = 16
N_PAD = 10240
RPT = N_PAD // NS
EP = 163840
ER = EP // 128
ER32 = ER // 32
ER16 = ER // 16
BN_EPS = 1e-5
BLK = 1000
NB = N // BLK

_mesh = plsc.VectorSubcoreMesh(core_axis_name="c", subcore_axis_name="s",
                               num_cores=NC, num_subcores=NS)

_LANE = [np.full((16,), l, np.int32) for l in range(16)]
_Z16 = np.zeros((16,), np.float32)


@functools.partial(
    pl.kernel,
    out_type=jax.ShapeDtypeStruct((NC, NS, RPT), jnp.float32),
    mesh=_mesh,
    scratch_types=[
        pltpu.VMEM((ER32, 128), jnp.int32),
        pltpu.VMEM((ER32, 128), jnp.float32),
        pltpu.VMEM((RPT,), jnp.float32),
        pltpu.VMEM_SHARED((N_PAD,), jnp.float32),
    ],
)
def _deg_kernel(dst_hbm, ew_hbm, out_hbm, idx_v, ew_v, zbuf, deg_sh):
    c = lax.axis_index("c")
    s = lax.axis_index("s")
    z16 = jnp.asarray(_Z16)

    def zfill(i, carry):
        zbuf[pl.ds(i * 16, 16)] = z16
        return carry

    lax.fori_loop(0, RPT // 16, zfill, 0)
    pltpu.sync_copy(zbuf, deg_sh.at[pl.ds(s * RPT, RPT)])
    plsc.subcore_barrier()

    base = (c * NS + s) * ER32
    pltpu.sync_copy(dst_hbm.at[pl.ds(base, ER32)], idx_v)
    pltpu.sync_copy(ew_hbm.at[pl.ds(base, ER32)], ew_v)

    def acc(j, carry):
        pltpu.sync_copy(ew_v.at[j], deg_sh.at[idx_v.at[j]], add=True)
        return carry

    lax.fori_loop(0, ER32, acc, 0)
    plsc.subcore_barrier()
    pltpu.sync_copy(deg_sh.at[pl.ds(s * RPT, RPT)], out_hbm.at[c, s])


@functools.partial(
    pl.kernel,
    out_type=jax.ShapeDtypeStruct((NC, NS, RPT, DH), jnp.float32),
    mesh=_mesh,
    scratch_types=[
        pltpu.VMEM((ER16, 128), jnp.int32),
        pltpu.VMEM((ER16, 128), jnp.int32),
        pltpu.VMEM((ER16, 128), jnp.int32),
        pltpu.VMEM((ER16, 128), jnp.float32),
        pltpu.VMEM((128, DH), jnp.float32),
        pltpu.VMEM_SHARED((N_PAD, DH), jnp.float32),
        pltpu.SemaphoreType.DMA,
    ],
)
def _msg_kernel(table_hbm, src_hbm, dst_hbm, ew_hbm, out_hbm,
                sraw, sidx, didx, ew_v, gbuf, acc_sh, gsem):
    c = lax.axis_index("c")
    s = lax.axis_index("s")
    z16 = jnp.asarray(_Z16)

    def zrow(r, carry):
        for v in range(8):
            gbuf[r, pl.ds(v * 16, 16)] = z16
        return carry

    lax.fori_loop(0, 128, zrow, 0)
    for k in range(RPT // 128):
        pltpu.sync_copy(gbuf, acc_sh.at[pl.ds(s * RPT + k * 128, 128)])

    base = s * ER16
    pltpu.sync_copy(src_hbm.at[pl.ds(base, ER16)], sraw)
    pltpu.sync_copy(dst_hbm.at[pl.ds(base, ER16)], didx)
    pltpu.sync_copy(ew_hbm.at[pl.ds(base, ER16)], ew_v)

    shift = c * N

    def sh(j, carry):
        for v in range(8):
            sl = pl.ds(v * 16, 16)
            sidx[j, sl] = sraw[j, sl] + shift
        return carry

    lax.fori_loop(0, ER16, sh, 0)
    plsc.subcore_barrier()

    def window(j, carry):
        pltpu.async_copy(table_hbm.at[sidx.at[j]], gbuf, gsem).wait()

        def grp(g, carry2):
            ev = ew_v[j, pl.ds(g * 16, 16)]
            for l in range(16):
                w = jnp.take(ev, jnp.asarray(_LANE[l]),
                             mode="promise_in_bounds")
                r = g * 16 + l
                for v in range(8):
                    sl = pl.ds(v * 16, 16)
                    gbuf[r, sl] = gbuf[r, sl] * w
            return carry2

        lax.fori_loop(0, 8, grp, 0)
        pltpu.sync_copy(gbuf, acc_sh.at[didx.at[j]], add=True)
        return carry

    lax.fori_loop(0, ER16, window, 0)
    plsc.subcore_barrier()
    pltpu.sync_copy(acc_sh.at[pl.ds(s * RPT, RPT)], out_hbm.at[c, s])


def _mm_body(h_ref, w_ref, deg_ref, out_ref):
    hw = jnp.dot(h_ref[...], w_ref[...], preferred_element_type=jnp.float32)
    deg = deg_ref[0, :] + deg_ref[1, :] + 1.0
    dis = lax.rsqrt(deg)
    out_ref[...] = hw * dis[:, None]


def _mm(h, w, deg2):
    return pl.pallas_call(
        _mm_body,
        grid=(NB, NC),
        in_specs=[
            pl.BlockSpec((BLK, D), lambda i, c: (i, 0)),
            pl.BlockSpec((D, DH), lambda i, c: (0, c)),
            pl.BlockSpec((NC, BLK), lambda i, c: (0, i)),
        ],
        out_specs=pl.BlockSpec((BLK, DH), lambda i, c: (c * NB + i, 0)),
        out_shape=jax.ShapeDtypeStruct((NC * N, DH), jnp.float32),
    )(h, w, deg2)


def _ep_body(msg_ref, hws0_ref, hws1_ref, deg_ref, x0_ref, b_ref,
             t2_ref, s1_ref, s2_ref):
    i = pl.program_id(0)
    deg = deg_ref[0, :] + deg_ref[1, :] + 1.0
    dis = lax.rsqrt(deg)[:, None]
    g0 = dis * (msg_ref[0] + hws0_ref[...])
    g1 = dis * (msg_ref[1] + hws1_ref[...])
    t = jnp.concatenate([g0, g1], axis=1) + b_ref[...]
    t2 = jnp.maximum(t, 0.0) + x0_ref[...]
    t2_ref[...] = t2
    ps1 = jnp.sum(t2, axis=0, keepdims=True)
    ps2 = jnp.sum(t2 * t2, axis=0, keepdims=True)

    @pl.when(i == 0)
    def _init():
        s1_ref[...] = ps1
        s2_ref[...] = ps2

    @pl.when(i != 0)
    def _accum():
        s1_ref[...] = s1_ref[...] + ps1
        s2_ref[...] = s2_ref[...] + ps2


def _ep(msg, hws, deg2, x0, b):
    return pl.pallas_call(
        _ep_body,
        grid=(NB,),
        in_specs=[
            pl.BlockSpec((NC, BLK, DH), lambda i: (0, i, 0)),
            pl.BlockSpec((BLK, DH), lambda i: (i, 0)),
            pl.BlockSpec((BLK, DH), lambda i: (NB + i, 0)),
            pl.BlockSpec((NC, BLK), lambda i: (0, i)),
            pl.BlockSpec((BLK, D), lambda i: (i, 0)),
            pl.BlockSpec((1, D), lambda i: (0, 0)),
        ],
        out_specs=[
            pl.BlockSpec((BLK, D), lambda i: (i, 0)),
            pl.BlockSpec((1, D), lambda i: (0, 0)),
            pl.BlockSpec((1, D), lambda i: (0, 0)),
        ],
        out_shape=[
            jax.ShapeDtypeStruct((N, D), jnp.float32),
            jax.ShapeDtypeStruct((1, D), jnp.float32),
            jax.ShapeDtypeStruct((1, D), jnp.float32),
        ],
    )(msg, hws, hws, deg2, x0, b)


def _bn_body(t2_ref, s1_ref, s2_ref, g_ref, bt_ref, out_ref):
    mean = s1_ref[...] * (1.0 / N)
    var = s2_ref[...] * (1.0 / N) - mean * mean
    rstd = lax.rsqrt(var + BN_EPS)
    out_ref[...] = (t2_ref[...] - mean) * rstd * g_ref[...] + bt_ref[...]


def _bn(t2, s1, s2, g, bt):
    return pl.pallas_call(
        _bn_body,
        grid=(NB,),
        in_specs=[
            pl.BlockSpec((BLK, D), lambda i: (i, 0)),
            pl.BlockSpec((1, D), lambda i: (0, 0)),
            pl.BlockSpec((1, D), lambda i: (0, 0)),
            pl.BlockSpec((1, D), lambda i: (0, 0)),
            pl.BlockSpec((1, D), lambda i: (0, 0)),
        ],
        out_specs=pl.BlockSpec((BLK, D), lambda i: (i, 0)),
        out_shape=jax.ShapeDtypeStruct((N, D), jnp.float32),
    )(t2, s1, s2, g, bt)


@jax.jit
def _run(x, src2, dst2, ew2, Ws, bs, gammas, betas):
    deg2 = _deg_kernel(dst2, ew2).reshape(NC, N_PAD)
    h = x
    for i in range(L):
        hws = _mm(h, Ws[i], deg2)
        msg = _msg_kernel(hws, src2, dst2, ew2).reshape(NC, N_PAD, DH)
        t2, s1, s2 = _ep(msg, hws, deg2, x, bs[i].reshape(1, D))
        h = _bn(t2, s1, s2, gammas[i].reshape(1, D), betas[i].reshape(1, D))
    return h


def kernel(x, edge_index, edge_weight, Ws, bs, gammas, betas):
    E = edge_weight.shape[0]
    pad = EP - E
    src = edge_index[0].astype(jnp.int32)
    dst = edge_index[1].astype(jnp.int32)
    padrows = N + (jnp.arange(pad, dtype=jnp.int32) % (N_PAD - N))
    src2 = jnp.concatenate([src, jnp.zeros((pad,), jnp.int32)]).reshape(ER, 128)
    dst2 = jnp.concatenate([dst, padrows]).reshape(ER, 128)
    ew2 = jnp.concatenate(
        [edge_weight.astype(jnp.float32), jnp.zeros((pad,), jnp.float32)]
    ).reshape(ER, 128)
    return _run(x, src2, dst2, ew2, Ws, bs, gammas, betas)


# speedup vs baseline: 5.3000x; 5.3000x over previous
import functools

import jax
import jax.numpy as jnp
from jax import lax
from jax.experimental import pallas as pl
from jax.experimental.pallas import tpu as pltpu
from jax.experimental.pallas import tpu_sc as plsc

N = 10000
D = 256
DH = 128
L = 3
NC = 2
NS = 16
N_PAD = 10240
RPT = N_PAD // NS
EP = 163840
ER = EP // 128
ER32 = ER // 32
ER16 = ER // 16
BN_EPS = 1e-5
BLK = 1000
NB = N // BLK


_GATHER_DNUMS = lax.GatherDimensionNumbers(
    offset_dims=(), collapsed_slice_dims=(0,), start_index_map=(0,))


def _splat(vec, lane):
    idx = jnp.full((16, 1), lane, jnp.int32)
    return lax.gather(vec, idx, _GATHER_DNUMS, slice_sizes=(1,),
                      mode=lax.GatherScatterMode.PROMISE_IN_BOUNDS)


def _sc_mesh():
    return plsc.VectorSubcoreMesh(core_axis_name="c", subcore_axis_name="s",
                                  num_cores=NC, num_subcores=NS)


@functools.cache
def _get_deg_kernel():
    return functools.partial(
        pl.kernel,
        out_type=jax.ShapeDtypeStruct((NS, RPT), jnp.float32),
        mesh=_sc_mesh(),
        scratch_types=[
            pltpu.VMEM((ER16, 128), jnp.int32),
            pltpu.VMEM((ER16, 128), jnp.float32),
            pltpu.VMEM((RPT,), jnp.float32),
            pltpu.VMEM_SHARED((N_PAD,), jnp.float32),
        ],
    )(_deg_body)


def _deg_body(dst_hbm, ew_hbm, out_hbm, idx_v, ew_v, zbuf, deg_sh):
    c = lax.axis_index("c")
    s = lax.axis_index("s")
    z16 = jnp.zeros((16,), jnp.float32)

    def zfill(i, carry):
        zbuf[pl.ds(i * 16, 16)] = z16
        return carry

    lax.fori_loop(0, RPT // 16, zfill, 0)
    pltpu.sync_copy(zbuf, deg_sh.at[pl.ds(s * RPT, RPT)])
    plsc.subcore_barrier()

    base = s * ER16
    pltpu.sync_copy(dst_hbm.at[pl.ds(base, ER16)], idx_v)
    pltpu.sync_copy(ew_hbm.at[pl.ds(base, ER16)], ew_v)

    def acc(j, carry):
        pltpu.sync_copy(ew_v.at[j], deg_sh.at[idx_v.at[j]], add=True)
        return carry

    lax.fori_loop(0, ER16, acc, 0)
    plsc.subcore_barrier()

    @pl.when(c == 0)
    def _write():
        pltpu.sync_copy(deg_sh.at[pl.ds(s * RPT, RPT)], out_hbm.at[s])


@functools.cache
def _get_msg_kernel():
    return functools.partial(
        pl.kernel,
        out_type=jax.ShapeDtypeStruct((NC, NS, RPT, DH), jnp.float32),
        mesh=_sc_mesh(),
        scratch_types=[
            pltpu.VMEM((ER16, 128), jnp.int32),
            pltpu.VMEM((ER16, 128), jnp.int32),
            pltpu.VMEM((ER16, 128), jnp.float32),
            pltpu.VMEM((128, DH), jnp.float32),
            pltpu.VMEM_SHARED((N_PAD, DH), jnp.float32),
            pltpu.SemaphoreType.DMA,
        ],
    )(_msg_body)


def _msg_body(table_hbm, src_hbm, dst_hbm, ew_hbm, out_hbm,
              sidx, didx, ew_v, gbuf, acc_sh, gsem):
    c = lax.axis_index("c")
    s = lax.axis_index("s")
    z16 = jnp.zeros((16,), jnp.float32)

    def zrow(r, carry):
        for v in range(8):
            gbuf[r, pl.ds(v * 16, 16)] = z16
        return carry

    lax.fori_loop(0, 128, zrow, 0)
    for k in range(RPT // 128):
        pltpu.sync_copy(gbuf, acc_sh.at[pl.ds(s * RPT + k * 128, 128)])

    base = s * ER16
    pltpu.sync_copy(src_hbm.at[pl.ds(base, ER16)], sidx)
    pltpu.sync_copy(dst_hbm.at[pl.ds(base, ER16)], didx)
    pltpu.sync_copy(ew_hbm.at[pl.ds(base, ER16)], ew_v)

    shift = c * N

    def sh(j, carry):
        for v in range(8):
            sl = pl.ds(v * 16, 16)
            sidx[j, sl] = sidx[j, sl] + shift
        return carry

    lax.fori_loop(0, ER16, sh, 0)
    plsc.subcore_barrier()

    def window(j, carry):
        pltpu.async_copy(table_hbm.at[sidx.at[j]], gbuf, gsem).wait()

        def grp(g, carry2):
            ev = ew_v[j, pl.ds(g * 16, 16)]
            for l in range(16):
                w = _splat(ev, l)
                r = g * 16 + l
                for v in range(8):
                    sl = pl.ds(v * 16, 16)
                    gbuf[r, sl] = gbuf[r, sl] * w
            return carry2

        lax.fori_loop(0, 8, grp, 0)
        pltpu.sync_copy(gbuf, acc_sh.at[didx.at[j]], add=True)
        return carry

    lax.fori_loop(0, ER16, window, 0)
    plsc.subcore_barrier()
    pltpu.sync_copy(acc_sh.at[pl.ds(s * RPT, RPT)], out_hbm.at[c, s])


def _mm_body(h_ref, w_ref, deg_ref, out_ref):
    hw = jnp.dot(h_ref[...], w_ref[...], preferred_element_type=jnp.float32)
    dis = lax.rsqrt(deg_ref[...] + 1.0)
    out_ref[...] = hw * dis


def _mm(h, w, deg2):
    return pl.pallas_call(
        _mm_body,
        grid=(NB, NC),
        in_specs=[
            pl.BlockSpec((BLK, D), lambda i, c: (i, 0)),
            pl.BlockSpec((D, DH), lambda i, c: (0, c)),
            pl.BlockSpec((BLK, 1), lambda i, c: (i, 0)),
        ],
        out_specs=pl.BlockSpec((BLK, DH), lambda i, c: (c * NB + i, 0)),
        out_shape=jax.ShapeDtypeStruct((NC * N, DH), jnp.float32),
    )(h, w, deg2)


def _ep_body(msg_ref, hws0_ref, hws1_ref, deg_ref, x0_ref, b_ref,
             t2_ref, s1_ref, s2_ref):
    i = pl.program_id(0)
    dis = lax.rsqrt(deg_ref[...] + 1.0)
    g0 = dis * (msg_ref[0] + hws0_ref[...])
    g1 = dis * (msg_ref[1] + hws1_ref[...])
    t = jnp.concatenate([g0, g1], axis=1) + b_ref[...]
    t2 = jnp.maximum(t, 0.0) + x0_ref[...]
    t2_ref[...] = t2
    ps1 = jnp.sum(t2, axis=0, keepdims=True)
    ps2 = jnp.sum(t2 * t2, axis=0, keepdims=True)

    @pl.when(i == 0)
    def _init():
        s1_ref[...] = ps1
        s2_ref[...] = ps2

    @pl.when(i != 0)
    def _accum():
        s1_ref[...] = s1_ref[...] + ps1
        s2_ref[...] = s2_ref[...] + ps2


def _ep(msg, hws, deg2, x0, b):
    return pl.pallas_call(
        _ep_body,
        grid=(NB,),
        in_specs=[
            pl.BlockSpec((NC, BLK, DH), lambda i: (0, i, 0)),
            pl.BlockSpec((BLK, DH), lambda i: (i, 0)),
            pl.BlockSpec((BLK, DH), lambda i: (NB + i, 0)),
            pl.BlockSpec((BLK, 1), lambda i: (i, 0)),
            pl.BlockSpec((BLK, D), lambda i: (i, 0)),
            pl.BlockSpec((1, D), lambda i: (0, 0)),
        ],
        out_specs=[
            pl.BlockSpec((BLK, D), lambda i: (i, 0)),
            pl.BlockSpec((1, D), lambda i: (0, 0)),
            pl.BlockSpec((1, D), lambda i: (0, 0)),
        ],
        out_shape=[
            jax.ShapeDtypeStruct((N, D), jnp.float32),
            jax.ShapeDtypeStruct((1, D), jnp.float32),
            jax.ShapeDtypeStruct((1, D), jnp.float32),
        ],
    )(msg, hws, hws, deg2, x0, b)


def _bn_body(t2_ref, s1_ref, s2_ref, g_ref, bt_ref, out_ref):
    mean = s1_ref[...] * (1.0 / N)
    var = s2_ref[...] * (1.0 / N) - mean * mean
    rstd = lax.rsqrt(var + BN_EPS)
    out_ref[...] = (t2_ref[...] - mean) * rstd * g_ref[...] + bt_ref[...]


def _bn(t2, s1, s2, g, bt):
    return pl.pallas_call(
        _bn_body,
        grid=(NB,),
        in_specs=[
            pl.BlockSpec((BLK, D), lambda i: (i, 0)),
            pl.BlockSpec((1, D), lambda i: (0, 0)),
            pl.BlockSpec((1, D), lambda i: (0, 0)),
            pl.BlockSpec((1, D), lambda i: (0, 0)),
            pl.BlockSpec((1, D), lambda i: (0, 0)),
        ],
        out_specs=pl.BlockSpec((BLK, D), lambda i: (i, 0)),
        out_shape=jax.ShapeDtypeStruct((N, D), jnp.float32),
    )(t2, s1, s2, g, bt)


@jax.jit
def _run(x, src2, dst2, ew2, Ws, bs, gammas, betas):
    deg1 = _get_deg_kernel()(dst2, ew2).reshape(N_PAD, 1)
    h = x
    for i in range(L):
        hws = _mm(h, Ws[i], deg1)
        msg = _get_msg_kernel()(hws, src2, dst2, ew2).reshape(NC, N_PAD, DH)
        t2, s1, s2 = _ep(msg, hws, deg1, x, bs[i].reshape(1, D))
        h = _bn(t2, s1, s2, gammas[i].reshape(1, D), betas[i].reshape(1, D))
    return h


def kernel(x, edge_index, edge_weight, Ws, bs, gammas, betas):
    E = edge_weight.shape[0]
    pad = EP - E
    src = edge_index[0].astype(jnp.int32)
    dst = edge_index[1].astype(jnp.int32)
    padrows = N + (jnp.arange(pad, dtype=jnp.int32) % (N_PAD - N))
    src2 = jnp.concatenate([src, jnp.zeros((pad,), jnp.int32)]).reshape(ER, 128)
    dst2 = jnp.concatenate([dst, padrows]).reshape(ER, 128)
    ew2 = jnp.concatenate(
        [edge_weight.astype(jnp.float32), jnp.zeros((pad,), jnp.float32)]
    ).reshape(ER, 128)
    return _run(x, src2, dst2, ew2, Ws, bs, gammas, betas)
